# Initial kernel scaffold; baseline (speedup 1.0000x reference)
import jax
import jax.numpy as jnp
from jax.experimental import pallas as pl


def kernel(boxes, scores):
    raise NotImplementedError("write your pallas kernel here")


# speedup vs baseline: 16.4620x; 16.4620x over previous
import jax
import jax.numpy as jnp
from jax.experimental import pallas as pl

_N = 20000
_LANES = 128
_ROWS = 160
_PAD = _ROWS * _LANES
_K = 1000
_MAX_DET = 300
_IOU_T = 0.5
_SCORE_T = 0.05


def _nms_body(s_ref, x1_ref, y1_ref, x2_ref, y2_ref, out_ref):
    s = s_ref[...]
    x1 = x1_ref[...]
    y1 = y1_ref[...]
    x2 = x2_ref[...]
    y2 = y2_ref[...]

    row_i = jax.lax.broadcasted_iota(jnp.int32, (_ROWS, _LANES), 0)
    col_i = jax.lax.broadcasted_iota(jnp.int32, (_ROWS, _LANES), 1)
    flat = row_i * _LANES + col_i
    pad = flat >= _N

    key = jnp.where(pad, jnp.int32(-1), jax.lax.bitcast_convert_type(s, jnp.int32))

    def bs_val(_, carry):
        lo, hi = carry
        mid = lo + (hi - lo) // 2
        c = jnp.sum((key >= mid).astype(jnp.int32))
        ok = c >= _K
        return jnp.where(ok, mid, lo), jnp.where(ok, hi, mid)

    v_lo, _ = jax.lax.fori_loop(
        0, 31, bs_val, (jnp.int32(0), jnp.int32(0x7F800000))
    )
    vstar = v_lo

    c_gt = jnp.sum((key > vstar).astype(jnp.int32))
    need = _K - c_gt
    tie = key == vstar

    def bs_idx(_, carry):
        lo, hi = carry
        mid = lo + (hi - lo) // 2
        g = jnp.sum((tie & (flat < mid)).astype(jnp.int32))
        ok = g >= need
        return jnp.where(ok, lo, mid), jnp.where(ok, hi, mid)

    _, i_hi = jax.lax.fori_loop(0, 16, bs_idx, (jnp.int32(0), jnp.int32(_PAD)))
    in_top = (key > vstar) | (tie & (flat < i_hi))

    valid0 = in_top & (s > _SCORE_T)
    areas = (x2 - x1) * (y2 - y1)
    lane = jax.lax.broadcasted_iota(jnp.int32, (1, _LANES), 1)
    zero = jnp.float32(0.0)

    live0 = jnp.where(valid0, s, -1.0)

    def step(t, live):
        m = jnp.max(live)
        any_valid = m >= 0.0
        cand = live == m
        isel = jnp.min(jnp.where(cand, flat, jnp.int32(0x7FFFFFFF)))
        sel = cand & (flat == isel)

        bx1 = jnp.sum(jnp.where(sel, x1, 0.0))
        by1 = jnp.sum(jnp.where(sel, y1, 0.0))
        bx2 = jnp.sum(jnp.where(sel, x2, 0.0))
        by2 = jnp.sum(jnp.where(sel, y2, 0.0))

        xx1 = jnp.maximum(bx1, x1)
        yy1 = jnp.maximum(by1, y1)
        xx2 = jnp.minimum(bx2, x2)
        yy2 = jnp.minimum(by2, y2)
        inter = jnp.maximum(xx2 - xx1, 0.0) * jnp.maximum(yy2 - yy1, 0.0)
        area_sel = (bx2 - bx1) * (by2 - by1)
        iou = inter / (area_sel + areas - inter + 1e-9)
        new_live = jnp.where(iou >= _IOU_T, -1.0, live)

        g = lambda v: jnp.where(any_valid, v, zero)
        row = (
            jnp.where(lane == 0, g(bx1), zero)
            + jnp.where(lane == 1, g(by1), zero)
            + jnp.where(lane == 2, g(bx2), zero)
            + jnp.where(lane == 3, g(by2), zero)
            + jnp.where(lane == 4, g(m), zero)
        )
        out_ref[pl.ds(t, 1), :] = row
        return new_live

    jax.lax.fori_loop(0, _MAX_DET, step, live0)


def kernel(boxes, scores):
    s = jnp.concatenate(
        [scores.astype(jnp.float32), jnp.full((_PAD - _N,), -1.0, jnp.float32)]
    ).reshape(_ROWS, _LANES)
    b = jnp.concatenate(
        [boxes.astype(jnp.float32), jnp.zeros((_PAD - _N, 4), jnp.float32)]
    )
    x1 = b[:, 0].reshape(_ROWS, _LANES)
    y1 = b[:, 1].reshape(_ROWS, _LANES)
    x2 = b[:, 2].reshape(_ROWS, _LANES)
    y2 = b[:, 3].reshape(_ROWS, _LANES)

    out = pl.pallas_call(
        _nms_body,
        out_shape=jax.ShapeDtypeStruct((_MAX_DET, _LANES), jnp.float32),
    )(s, x1, y1, x2, y2)
    return out[:, :5]
